# baseline (speedup 1.0000x reference)
import functools

import jax
import jax.numpy as jnp
from jax import lax
from jax.experimental import pallas as pl
from jax.experimental.pallas import tpu as pltpu
from jax.experimental.pallas import tpu_sc as plsc

N = 10000
DEG = 16
E = N * DEG
H = 128
NC_SC = 2
NS_SC = 16
NW = NC_SC * NS_SC

_pallas_call = pl.pallas_call


@functools.lru_cache(maxsize=None)
def _make_sc_gather(M, C, dtype, W):
    npw = M // (NW * C)
    assert npw * NW * C == M
    mesh = plsc.VectorSubcoreMesh(core_axis_name="c", subcore_axis_name="s")

    rpw = npw * C
    NB = 3

    def body(table_hbm, idx_hbm, out_hbm, idx_v, *bufsem):
        bufs = bufsem[:NB]
        sgs = bufsem[NB:2 * NB]
        sos = bufsem[2 * NB:]
        wid = lax.axis_index("s") * NC_SC + lax.axis_index("c")
        chunk0 = wid * npw
        pltpu.sync_copy(idx_hbm.at[pl.ds(wid * rpw, rpw)], idx_v)
        hg = [None] * NB
        ho = [None] * NB
        for j in range(min(NB - 1, npw)):
            hg[j] = pltpu.async_copy(
                table_hbm.at[idx_v.at[pl.ds(j * C, C)]], bufs[j], sgs[j])
        for i in range(npw):
            b = i % NB
            hg[b].wait()
            ho[b] = pltpu.async_copy(
                bufs[b], out_hbm.at[pl.ds((chunk0 + i) * C, C)], sos[b])
            j = i + NB - 1
            if j < npw:
                bj = j % NB
                if ho[bj] is not None:
                    ho[bj].wait()
                    ho[bj] = None
                hg[bj] = pltpu.async_copy(
                    table_hbm.at[idx_v.at[pl.ds(j * C, C)]], bufs[bj], sgs[bj])
        for b in range(NB):
            if ho[b] is not None:
                ho[b].wait()

    return pl.kernel(
        body,
        out_type=jax.ShapeDtypeStruct((M, W), dtype),
        mesh=mesh,
        scratch_types=(
            [pltpu.VMEM((rpw,), jnp.int32)] +
            [pltpu.VMEM((C, W), dtype) for _ in range(NB)] +
            [pltpu.SemaphoreType.DMA for _ in range(2 * NB)]
        ),
    )


def _gather_rows(table, idx_flat, M, C):
    return _make_sc_gather(M, C, table.dtype, H)(table, idx_flat)


CN_SUM = 16


@functools.lru_cache(maxsize=None)
def _make_sc_gather_sum(N_PAD_):
    nodes_pw = N_PAD_ // NW
    npw = nodes_pw // CN_SUM
    C = CN_SUM * DEG
    mesh = plsc.VectorSubcoreMesh(core_axis_name="c", subcore_axis_name="s")

    def body(table_hbm, idx_hbm, dst_hbm, zeros_hbm, out_hbm, idx_v, *rest):
        dsts = rest[:npw]
        buf0, buf1, shared, sg0, sg1, so0, so1 = rest[npw:]
        bufs = (buf0, buf1)
        sgs = (sg0, sg1)
        sos = (so0, so1)
        wid = lax.axis_index("s") * NC_SC + lax.axis_index("c")
        base = wid * nodes_pw
        rpw = nodes_pw * DEG
        pltpu.sync_copy(idx_hbm.at[pl.ds(wid * rpw, rpw)], idx_v)
        for i in range(npw):
            pltpu.sync_copy(dst_hbm.at[pl.ds(wid * rpw + i * C, C)], dsts[i])
        hg = [None, None]
        ho = [None, None]
        hg[0] = pltpu.async_copy(
            table_hbm.at[idx_v.at[pl.ds(0, C)]], bufs[0], sgs[0])
        for i in range(npw):
            b = i % 2
            if i + 1 < npw:
                hg[(i + 1) % 2] = pltpu.async_copy(
                    table_hbm.at[idx_v.at[pl.ds((i + 1) * C, C)]],
                    bufs[(i + 1) % 2], sgs[(i + 1) % 2])
            lbase = lax.axis_index("s") * nodes_pw
            pltpu.sync_copy(zeros_hbm,
                            shared.at[pl.ds(lbase + i * CN_SUM, CN_SUM)])
            hg[b].wait()
            pltpu.sync_copy(bufs[b], shared.at[dsts[i]], add=True)
            ho[b] = pltpu.async_copy(
                shared.at[pl.ds(lbase + i * CN_SUM, CN_SUM)],
                out_hbm.at[pl.ds(base + i * CN_SUM, CN_SUM)], sos[b])
        for b in range(2):
            if ho[b] is not None:
                ho[b].wait()

    return pl.kernel(
        body,
        out_type=jax.ShapeDtypeStruct((N_PAD_, H), jnp.float32),
        mesh=mesh,
        scratch_types=(
            [pltpu.VMEM((nodes_pw * DEG,), jnp.int32)] +
            [pltpu.VMEM((C,), jnp.int32) for _ in range(npw)] +
            [pltpu.VMEM((C, H), jnp.float32) for _ in range(2)] +
            [pltpu.VMEM_SHARED((N_PAD_ // NC_SC, H), jnp.float32)] +
            [pltpu.SemaphoreType.DMA for _ in range(4)]
        ),
    )


def _gather_sum(table, idx_node_major, dst_abs, zeros16, n_pad):
    return _make_sc_gather_sum(n_pad)(table, idx_node_major, dst_abs, zeros16)


def _dot(a, b):
    return jnp.dot(a, b, preferred_element_type=jnp.float32)


def _ln(x, g, b):
    mu = jnp.mean(x, axis=-1, keepdims=True)
    d = x - mu
    var = jnp.mean(d * d, axis=-1, keepdims=True)
    return d * lax.rsqrt(var + 1e-5) * g + b


def _leaky(x):
    return jnp.where(x >= 0, x, 0.01 * x)


TILE = 400
GRID = N // TILE


def _transform_body(hg2_ref, hdur_ref, wt_ref, b_ref, out_ref):
    out_ref[...] = (_dot(hg2_ref[...], wt_ref[:H]) +
                    _dot(hdur_ref[...], wt_ref[H:]) + b_ref[...])


def _transform(h_g2, h_dur, wt, b):
    return _pallas_call(
        _transform_body,
        grid=(GRID,),
        in_specs=[
            pl.BlockSpec((TILE, H), lambda i: (i, 0)),
            pl.BlockSpec((TILE, H), lambda i: (i, 0)),
            pl.BlockSpec((2 * H, H), lambda i: (0, 0)),
            pl.BlockSpec((1, H), lambda i: (0, 0)),
        ],
        out_specs=pl.BlockSpec((TILE, H), lambda i: (i, 0)),
        out_shape=jax.ShapeDtypeStruct((N, H), jnp.float32),
    )(h_g2, h_dur, wt, b)


def _gru_math(s_ref, hh, weT_ref, wiT_ref, whT_ref, bi_ref, bh_ref):
    a = _dot(s_ref[...], weT_ref[...])
    gi = _dot(a, wiT_ref[...]) + bi_ref[...]
    gh = _dot(hh, whT_ref[...]) + bh_ref[...]
    ir, iz, inn = gi[:, :H], gi[:, H:2 * H], gi[:, 2 * H:]
    hr, hz, hn2 = gh[:, :H], gh[:, H:2 * H], gh[:, 2 * H:]
    rg = jax.nn.sigmoid(ir + hr)
    zg = jax.nn.sigmoid(iz + hz)
    ng = jnp.tanh(inn + rg * hn2)
    return (1.0 - zg) * ng + zg * hh


def _ggc_step_body(s_ref, hh_ref, weT_ref, wiT_ref, whT_ref, bi_ref, bh_ref,
                   out_ref):
    out_ref[...] = _gru_math(s_ref, hh_ref[...], weT_ref, wiT_ref, whT_ref,
                             bi_ref, bh_ref)


def _ggc_step(S, hh, weT, wiT, whT, bi, bh):
    return _pallas_call(
        _ggc_step_body,
        grid=(GRID,),
        in_specs=[
            pl.BlockSpec((TILE, H), lambda i: (i, 0)),
            pl.BlockSpec((TILE, H), lambda i: (i, 0)),
            pl.BlockSpec((H, H), lambda i: (0, 0)),
            pl.BlockSpec((H, 3 * H), lambda i: (0, 0)),
            pl.BlockSpec((H, 3 * H), lambda i: (0, 0)),
            pl.BlockSpec((1, 3 * H), lambda i: (0, 0)),
            pl.BlockSpec((1, 3 * H), lambda i: (0, 0)),
        ],
        out_specs=pl.BlockSpec((TILE, H), lambda i: (i, 0)),
        out_shape=jax.ShapeDtypeStruct((N, H), jnp.float32),
    )(S, hh, weT, wiT, whT, bi, bh)


def _ggc_final_body(s_ref, hh_ref, res_ref, weT_ref, wiT_ref, whT_ref,
                    bi_ref, bh_ref, lng_ref, lnb_ref, mean_ref):
    hh2 = _gru_math(s_ref, hh_ref[...], weT_ref, wiT_ref, whT_ref, bi_ref,
                    bh_ref)
    v = _leaky(_ln(hh2 + res_ref[...], lng_ref[...], lnb_ref[...]))

    @pl.when(pl.program_id(0) == 0)
    def _():
        mean_ref[...] = jnp.zeros_like(mean_ref)

    mean_ref[...] += jnp.sum(v, axis=0, keepdims=True) * (1.0 / N)


def _ggc_final(S, hh, res, weT, wiT, whT, bi, bh, lng, lnb):
    return _pallas_call(
        _ggc_final_body,
        grid=(GRID,),
        in_specs=[
            pl.BlockSpec((TILE, H), lambda i: (i, 0)),
            pl.BlockSpec((TILE, H), lambda i: (i, 0)),
            pl.BlockSpec((TILE, H), lambda i: (i, 0)),
            pl.BlockSpec((H, H), lambda i: (0, 0)),
            pl.BlockSpec((H, 3 * H), lambda i: (0, 0)),
            pl.BlockSpec((H, 3 * H), lambda i: (0, 0)),
            pl.BlockSpec((1, 3 * H), lambda i: (0, 0)),
            pl.BlockSpec((1, 3 * H), lambda i: (0, 0)),
            pl.BlockSpec((1, H), lambda i: (0, 0)),
            pl.BlockSpec((1, H), lambda i: (0, 0)),
        ],
        out_specs=pl.BlockSpec((1, H), lambda i: (0, 0)),
        out_shape=jax.ShapeDtypeStruct((1, H), jnp.float32),
    )(S, hh, res, weT, wiT, whT, bi, bh, lng, lnb)


def _lstm_sage_body(g_ref, fd_ref, wg_ref, bg_ref, wselfT_ref,
                    wneighT_ref, sb_ref, n1g_ref, n1b_ref, n3g_ref, n3b_ref,
                    out_ref, mean_ref):
    fd = fd_ref[...]
    w = wg_ref[...]
    bg = bg_ref[...]
    h = jnp.zeros((TILE, H), jnp.float32)
    c = jnp.zeros((TILE, H), jnp.float32)
    for t in range(DEG):
        xh = jnp.concatenate(
            [g_ref[t].astype(jnp.bfloat16), h.astype(jnp.bfloat16)], axis=1)
        gates = _dot(xh, w) + bg
        i_ = gates[:, :H]
        f_ = gates[:, H:2 * H]
        g_ = gates[:, 2 * H:3 * H]
        o_ = gates[:, 3 * H:]
        c = jax.nn.sigmoid(f_) * c + jax.nn.sigmoid(i_) * jnp.tanh(g_)
        h = jax.nn.sigmoid(o_) * jnp.tanh(c)
    conv = _dot(fd, wselfT_ref[...]) + _dot(h, wneighT_ref[...]) + sb_ref[...]
    v = _leaky(_ln(conv, n1g_ref[...], n1b_ref[...]))
    v = fd + v
    v = _leaky(_ln(v, n3g_ref[...], n3b_ref[...]))
    out_ref[...] = v

    @pl.when(pl.program_id(0) == 0)
    def _():
        mean_ref[...] = jnp.zeros_like(mean_ref)

    mean_ref[...] += jnp.sum(v, axis=0, keepdims=True) * (1.0 / N)


def _lstm_sage(G, fd, wg, bg, wselfT, wneighT, sb, n1g, n1b, n3g, n3b):
    return _pallas_call(
        _lstm_sage_body,
        grid=(GRID,),
        in_specs=[
            pl.BlockSpec((DEG, TILE, H), lambda i: (0, i, 0)),
            pl.BlockSpec((TILE, H), lambda i: (i, 0)),
            pl.BlockSpec((2 * H, 4 * H), lambda i: (0, 0)),
            pl.BlockSpec((1, 4 * H), lambda i: (0, 0)),
            pl.BlockSpec((H, H), lambda i: (0, 0)),
            pl.BlockSpec((H, H), lambda i: (0, 0)),
            pl.BlockSpec((1, H), lambda i: (0, 0)),
            pl.BlockSpec((1, H), lambda i: (0, 0)),
            pl.BlockSpec((1, H), lambda i: (0, 0)),
            pl.BlockSpec((1, H), lambda i: (0, 0)),
            pl.BlockSpec((1, H), lambda i: (0, 0)),
        ],
        out_specs=[
            pl.BlockSpec((TILE, H), lambda i: (i, 0)),
            pl.BlockSpec((1, H), lambda i: (0, 0)),
        ],
        out_shape=[
            jax.ShapeDtypeStruct((N, H), jnp.float32),
            jax.ShapeDtypeStruct((1, H), jnp.float32),
        ],
    )(G, fd, wg, bg, wselfT, wneighT, sb, n1g, n1b, n3g, n3b)


def _head_body(hg2m_ref, actm_ref, durm_ref, mhW1T_ref, mhb1_ref, mhW2T_ref,
               mhb2_ref, mtW1T_ref, mtb1_ref, mtW2T_ref, mtb2_ref, mlpW1T_ref,
               mlpb1_ref, mlpW2T_ref, mlpb2_ref, clsWT_ref, clsb_ref, out_ref):
    hg2m = hg2m_ref[...]
    hgm = actm_ref[...] + durm_ref[...]
    out1 = jax.nn.relu(_dot(jax.nn.relu(_dot(hg2m, mhW1T_ref[...]) + mhb1_ref[...]),
                            mhW2T_ref[...]) + mhb2_ref[...])
    out2 = jax.nn.relu(_dot(jax.nn.relu(_dot(hgm, mtW1T_ref[...]) + mtb1_ref[...]),
                            mtW2T_ref[...]) + mtb2_ref[...])
    w = mlpW1T_ref[...]
    pre = (_dot(out1, w[:H]) + _dot(hg2m, w[H:2 * H]) +
           _dot(out2, w[2 * H:3 * H]) + _dot(hgm, w[3 * H:]) + mlpb1_ref[...])
    preds = jax.nn.relu(_dot(jax.nn.relu(pre), mlpW2T_ref[...]) + mlpb2_ref[...])
    out_ref[...] = _dot(preds, clsWT_ref[...]) + clsb_ref[...]


def _head(hg2m, actm, durm, mhW1T, mhb1, mhW2T, mhb2, mtW1T, mtb1, mtW2T,
          mtb2, mlpW1T, mlpb1, mlpW2T, mlpb2, clsWT, clsb):
    return _pallas_call(
        _head_body,
        out_shape=jax.ShapeDtypeStruct((1, H), jnp.float32),
    )(hg2m, actm, durm, mhW1T, mhb1, mhW2T, mhb2, mtW1T, mtb1, mtW2T, mtb2,
      mlpW1T, mlpb1, mlpW2T, mlpb2, clsWT, clsb)


C_EDGE = 200
C_EMB = 120
N_PAD = 10240
EP = DEG * N_PAD


def kernel(activity_ids, duration_ids, g2_activity_ids, src_a2d, src_d2a,
           g2_src, emb_activity, emb_duration, emb_activity2, transform_W,
           transform_b, lstm_Wih, lstm_Whh, lstm_bih, lstm_bhh, fc_self_W,
           fc_neigh_W, sage_b, norm1_g, norm1_b, norm3_g, norm3_b, ggc_We,
           gru_Wi, gru_Wh, gru_bi, gru_bh, homo_ln_g, homo_ln_b, mh_W1, mh_b1,
           mh_W2, mh_b2, mt_W1, mt_b1, mt_W2, mt_b2, mlp_W1, mlp_b1, mlp_W2,
           mlp_b2, cls_W, cls_b):
    f32 = jnp.float32

    def prep_edge_idx(src):
        return src.astype(jnp.int32).reshape(N, DEG).T.reshape(E)

    def pad_ids(ids, off):
        return jnp.pad(ids.astype(jnp.int32), (0, N_PAD - N)) + off

    emb_table = jnp.concatenate([emb_activity, emb_duration, emb_activity2], axis=0)
    emb_idx = jnp.concatenate([
        pad_ids(activity_ids, 0),
        pad_ids(duration_ids, 1000),
        pad_ids(g2_activity_ids, 2000),
    ])
    emb_out = _gather_rows(emb_table, emb_idx, 3 * N_PAD, C_EMB)
    h_act = emb_out[0:N]
    h_dur = emb_out[N_PAD:N_PAD + N]
    h_g2 = emb_out[2 * N_PAD:2 * N_PAD + N]

    bf16 = jnp.bfloat16
    a2d_idx = prep_edge_idx(src_a2d)
    d2a_idx = prep_edge_idx(src_d2a)
    weT = ggc_We.T.astype(f32)
    wiT = gru_Wi.T
    whT = gru_Wh.T
    bi = gru_bi.reshape(1, 3 * H)
    bh = gru_bh.reshape(1, 3 * H)

    def sage(G, fd, l, r, ti):
        return _lstm_sage(
            G, fd,
            jnp.concatenate([lstm_Wih[l, r].T, lstm_Whh[l, r].T],
                            axis=0).astype(bf16),
            (lstm_bih[l, r] + lstm_bhh[l, r]).reshape(1, 4 * H),
            fc_self_W[l, r].T, fc_neigh_W[l, r].T,
            sage_b[l, r].reshape(1, H),
            norm1_g[l, ti].reshape(1, H), norm1_b[l, ti].reshape(1, H),
            norm3_g[l, ti].reshape(1, H), norm3_b[l, ti].reshape(1, H))

    h_new = _transform(h_g2, h_dur, transform_W.T, transform_b.reshape(1, H))
    g2_nm = jnp.pad(g2_src.astype(jnp.int32).reshape(N, DEG),
                    ((0, N_PAD - N), (0, 0))).reshape(EP)
    rpw_s = (N_PAD // NW) * DEG
    gr = jnp.arange(EP, dtype=jnp.int32)
    dst_abs = ((gr // rpw_s) // NC_SC) * (N_PAD // NW) + (gr % rpw_s) // DEG
    zeros16 = jnp.zeros((CN_SUM, H), jnp.float32)
    S1 = _gather_sum(h_new, g2_nm, dst_abs, zeros16, N_PAD)
    Ga0 = _gather_rows(h_act, a2d_idx, E, C_EDGE).reshape(DEG, N, H)
    Gd0 = _gather_rows(h_dur, d2a_idx, E, C_EDGE).reshape(DEG, N, H)
    hh1 = _ggc_step(S1, h_new, weT, wiT, whT, bi, bh)
    S2 = _gather_sum(hh1, g2_nm, dst_abs, zeros16, N_PAD)
    new_dur, _ = sage(Ga0, h_dur, 0, 0, 1)
    new_act, _ = sage(Gd0, h_act, 0, 1, 0)
    Gd1 = _gather_rows(new_dur, d2a_idx, E, C_EDGE).reshape(DEG, N, H)
    Ga1 = _gather_rows(new_act, a2d_idx, E, C_EDGE).reshape(DEG, N, H)
    hg2_mean = _ggc_final(S2, hh1, h_new, weT, wiT, whT, bi, bh,
                          homo_ln_g.reshape(1, H), homo_ln_b.reshape(1, H))
    _, dur_mean = sage(Ga1, new_dur, 1, 0, 1)
    _, act_mean = sage(Gd1, new_act, 1, 1, 0)

    clsWT = jnp.zeros((H, H), f32).at[:, :16].set(cls_W.T)
    clsb = jnp.zeros((1, H), f32).at[0, :16].set(cls_b)
    logits = _head(
        hg2_mean, act_mean, dur_mean,
        mh_W1.T, mh_b1.reshape(1, H), mh_W2.T, mh_b2.reshape(1, H),
        mt_W1.T, mt_b1.reshape(1, H), mt_W2.T, mt_b2.reshape(1, H),
        mlp_W1.T, mlp_b1.reshape(1, 2 * H), mlp_W2.T, mlp_b2.reshape(1, H),
        clsWT, clsb)
    return logits[:, :16]


# speedup vs baseline: 1.0039x; 1.0039x over previous
import functools

import jax
import jax.numpy as jnp
from jax import lax
from jax.experimental import pallas as pl
from jax.experimental.pallas import tpu as pltpu
from jax.experimental.pallas import tpu_sc as plsc

N = 10000
DEG = 16
E = N * DEG
H = 128
NC_SC = 2
NS_SC = 16
NW = NC_SC * NS_SC

_pallas_call = pl.pallas_call


@functools.lru_cache(maxsize=None)
def _make_sc_gather(M, C, dtype, W):
    npw = M // (NW * C)
    assert npw * NW * C == M
    mesh = plsc.VectorSubcoreMesh(core_axis_name="c", subcore_axis_name="s")

    rpw = npw * C
    NB = 3

    def body(table_hbm, idx_hbm, out_hbm, idx_v, *bufsem):
        bufs = bufsem[:NB]
        sgs = bufsem[NB:2 * NB]
        sos = bufsem[2 * NB:]
        wid = lax.axis_index("s") * NC_SC + lax.axis_index("c")
        chunk0 = wid * npw
        pltpu.sync_copy(idx_hbm.at[pl.ds(wid * rpw, rpw)], idx_v)
        hg = [None] * NB
        ho = [None] * NB
        for j in range(min(NB - 1, npw)):
            hg[j] = pltpu.async_copy(
                table_hbm.at[idx_v.at[pl.ds(j * C, C)]], bufs[j], sgs[j])
        for i in range(npw):
            b = i % NB
            hg[b].wait()
            ho[b] = pltpu.async_copy(
                bufs[b], out_hbm.at[pl.ds((chunk0 + i) * C, C)], sos[b])
            j = i + NB - 1
            if j < npw:
                bj = j % NB
                if ho[bj] is not None:
                    ho[bj].wait()
                    ho[bj] = None
                hg[bj] = pltpu.async_copy(
                    table_hbm.at[idx_v.at[pl.ds(j * C, C)]], bufs[bj], sgs[bj])
        for b in range(NB):
            if ho[b] is not None:
                ho[b].wait()

    return pl.kernel(
        body,
        out_type=jax.ShapeDtypeStruct((M, W), dtype),
        mesh=mesh,
        scratch_types=(
            [pltpu.VMEM((rpw,), jnp.int32)] +
            [pltpu.VMEM((C, W), dtype) for _ in range(NB)] +
            [pltpu.SemaphoreType.DMA for _ in range(2 * NB)]
        ),
    )


def _gather_rows(table, idx_flat, M, C):
    return _make_sc_gather(M, C, table.dtype, H)(table, idx_flat)


CN_SUM = 16


@functools.lru_cache(maxsize=None)
def _make_sc_gather_sum(N_PAD_):
    nodes_pw = N_PAD_ // NW
    npw = nodes_pw // CN_SUM
    C = CN_SUM * DEG
    mesh = plsc.VectorSubcoreMesh(core_axis_name="c", subcore_axis_name="s")

    def body(table_hbm, idx_hbm, dst_hbm, zeros_hbm, out_hbm, idx_v, *rest):
        dsts = rest[:npw]
        buf0, buf1, shared, sg0, sg1, so0, so1 = rest[npw:]
        bufs = (buf0, buf1)
        sgs = (sg0, sg1)
        sos = (so0, so1)
        wid = lax.axis_index("s") * NC_SC + lax.axis_index("c")
        base = wid * nodes_pw
        rpw = nodes_pw * DEG
        pltpu.sync_copy(idx_hbm.at[pl.ds(wid * rpw, rpw)], idx_v)
        for i in range(npw):
            pltpu.sync_copy(dst_hbm.at[pl.ds(wid * rpw + i * C, C)], dsts[i])
        lbase = lax.axis_index("s") * nodes_pw
        pltpu.sync_copy(zeros_hbm, shared.at[pl.ds(lbase, nodes_pw)])
        hg = [None, None]
        ho = [None, None]
        hg[0] = pltpu.async_copy(
            table_hbm.at[idx_v.at[pl.ds(0, C)]], bufs[0], sgs[0])
        for i in range(npw):
            b = i % 2
            if i + 1 < npw:
                hg[(i + 1) % 2] = pltpu.async_copy(
                    table_hbm.at[idx_v.at[pl.ds((i + 1) * C, C)]],
                    bufs[(i + 1) % 2], sgs[(i + 1) % 2])
            hg[b].wait()
            pltpu.sync_copy(bufs[b], shared.at[dsts[i]], add=True)
            ho[b] = pltpu.async_copy(
                shared.at[pl.ds(lbase + i * CN_SUM, CN_SUM)],
                out_hbm.at[pl.ds(base + i * CN_SUM, CN_SUM)], sos[b])
        for b in range(2):
            if ho[b] is not None:
                ho[b].wait()

    return pl.kernel(
        body,
        out_type=jax.ShapeDtypeStruct((N_PAD_, H), jnp.float32),
        mesh=mesh,
        scratch_types=(
            [pltpu.VMEM((nodes_pw * DEG,), jnp.int32)] +
            [pltpu.VMEM((C,), jnp.int32) for _ in range(npw)] +
            [pltpu.VMEM((C, H), jnp.float32) for _ in range(2)] +
            [pltpu.VMEM_SHARED((N_PAD_ // NC_SC, H), jnp.float32)] +
            [pltpu.SemaphoreType.DMA for _ in range(4)]
        ),
    )


def _gather_sum(table, idx_node_major, dst_abs, zeros16, n_pad):
    return _make_sc_gather_sum(n_pad)(table, idx_node_major, dst_abs, zeros16)


def _dot(a, b):
    return jnp.dot(a, b, preferred_element_type=jnp.float32)


def _ln(x, g, b):
    mu = jnp.mean(x, axis=-1, keepdims=True)
    d = x - mu
    var = jnp.mean(d * d, axis=-1, keepdims=True)
    return d * lax.rsqrt(var + 1e-5) * g + b


def _leaky(x):
    return jnp.where(x >= 0, x, 0.01 * x)


TILE = 400
GRID = N // TILE


def _transform_body(hg2_ref, hdur_ref, wt_ref, b_ref, out_ref):
    out_ref[...] = (_dot(hg2_ref[...], wt_ref[:H]) +
                    _dot(hdur_ref[...], wt_ref[H:]) + b_ref[...])


def _transform(h_g2, h_dur, wt, b):
    return _pallas_call(
        _transform_body,
        grid=(GRID,),
        in_specs=[
            pl.BlockSpec((TILE, H), lambda i: (i, 0)),
            pl.BlockSpec((TILE, H), lambda i: (i, 0)),
            pl.BlockSpec((2 * H, H), lambda i: (0, 0)),
            pl.BlockSpec((1, H), lambda i: (0, 0)),
        ],
        out_specs=pl.BlockSpec((TILE, H), lambda i: (i, 0)),
        out_shape=jax.ShapeDtypeStruct((N, H), jnp.float32),
    )(h_g2, h_dur, wt, b)


def _gru_math(s_ref, hh, weT_ref, wiT_ref, whT_ref, bi_ref, bh_ref):
    a = _dot(s_ref[...], weT_ref[...])
    gi = _dot(a, wiT_ref[...]) + bi_ref[...]
    gh = _dot(hh, whT_ref[...]) + bh_ref[...]
    ir, iz, inn = gi[:, :H], gi[:, H:2 * H], gi[:, 2 * H:]
    hr, hz, hn2 = gh[:, :H], gh[:, H:2 * H], gh[:, 2 * H:]
    rg = jax.nn.sigmoid(ir + hr)
    zg = jax.nn.sigmoid(iz + hz)
    ng = jnp.tanh(inn + rg * hn2)
    return (1.0 - zg) * ng + zg * hh


def _ggc_step_body(s_ref, hh_ref, weT_ref, wiT_ref, whT_ref, bi_ref, bh_ref,
                   out_ref):
    out_ref[...] = _gru_math(s_ref, hh_ref[...], weT_ref, wiT_ref, whT_ref,
                             bi_ref, bh_ref)


def _ggc_step(S, hh, weT, wiT, whT, bi, bh):
    return _pallas_call(
        _ggc_step_body,
        grid=(GRID,),
        in_specs=[
            pl.BlockSpec((TILE, H), lambda i: (i, 0)),
            pl.BlockSpec((TILE, H), lambda i: (i, 0)),
            pl.BlockSpec((H, H), lambda i: (0, 0)),
            pl.BlockSpec((H, 3 * H), lambda i: (0, 0)),
            pl.BlockSpec((H, 3 * H), lambda i: (0, 0)),
            pl.BlockSpec((1, 3 * H), lambda i: (0, 0)),
            pl.BlockSpec((1, 3 * H), lambda i: (0, 0)),
        ],
        out_specs=pl.BlockSpec((TILE, H), lambda i: (i, 0)),
        out_shape=jax.ShapeDtypeStruct((N, H), jnp.float32),
    )(S, hh, weT, wiT, whT, bi, bh)


def _ggc_final_body(s_ref, hh_ref, res_ref, weT_ref, wiT_ref, whT_ref,
                    bi_ref, bh_ref, lng_ref, lnb_ref, mean_ref):
    hh2 = _gru_math(s_ref, hh_ref[...], weT_ref, wiT_ref, whT_ref, bi_ref,
                    bh_ref)
    v = _leaky(_ln(hh2 + res_ref[...], lng_ref[...], lnb_ref[...]))

    @pl.when(pl.program_id(0) == 0)
    def _():
        mean_ref[...] = jnp.zeros_like(mean_ref)

    mean_ref[...] += jnp.sum(v, axis=0, keepdims=True) * (1.0 / N)


def _ggc_final(S, hh, res, weT, wiT, whT, bi, bh, lng, lnb):
    return _pallas_call(
        _ggc_final_body,
        grid=(GRID,),
        in_specs=[
            pl.BlockSpec((TILE, H), lambda i: (i, 0)),
            pl.BlockSpec((TILE, H), lambda i: (i, 0)),
            pl.BlockSpec((TILE, H), lambda i: (i, 0)),
            pl.BlockSpec((H, H), lambda i: (0, 0)),
            pl.BlockSpec((H, 3 * H), lambda i: (0, 0)),
            pl.BlockSpec((H, 3 * H), lambda i: (0, 0)),
            pl.BlockSpec((1, 3 * H), lambda i: (0, 0)),
            pl.BlockSpec((1, 3 * H), lambda i: (0, 0)),
            pl.BlockSpec((1, H), lambda i: (0, 0)),
            pl.BlockSpec((1, H), lambda i: (0, 0)),
        ],
        out_specs=pl.BlockSpec((1, H), lambda i: (0, 0)),
        out_shape=jax.ShapeDtypeStruct((1, H), jnp.float32),
    )(S, hh, res, weT, wiT, whT, bi, bh, lng, lnb)


def _lstm_sage_body(g_ref, fd_ref, wg_ref, bg_ref, wselfT_ref,
                    wneighT_ref, sb_ref, n1g_ref, n1b_ref, n3g_ref, n3b_ref,
                    out_ref, mean_ref):
    fd = fd_ref[...]
    w = wg_ref[...]
    bg = bg_ref[...]
    h = jnp.zeros((TILE, H), jnp.float32)
    c = jnp.zeros((TILE, H), jnp.float32)
    for t in range(DEG):
        xh = jnp.concatenate(
            [g_ref[t].astype(jnp.bfloat16), h.astype(jnp.bfloat16)], axis=1)
        gates = _dot(xh, w) + bg
        i_ = gates[:, :H]
        f_ = gates[:, H:2 * H]
        g_ = gates[:, 2 * H:3 * H]
        o_ = gates[:, 3 * H:]
        c = jax.nn.sigmoid(f_) * c + jax.nn.sigmoid(i_) * jnp.tanh(g_)
        h = jax.nn.sigmoid(o_) * jnp.tanh(c)
    conv = _dot(fd, wselfT_ref[...]) + _dot(h, wneighT_ref[...]) + sb_ref[...]
    v = _leaky(_ln(conv, n1g_ref[...], n1b_ref[...]))
    v = fd + v
    v = _leaky(_ln(v, n3g_ref[...], n3b_ref[...]))
    out_ref[...] = v

    @pl.when(pl.program_id(0) == 0)
    def _():
        mean_ref[...] = jnp.zeros_like(mean_ref)

    mean_ref[...] += jnp.sum(v, axis=0, keepdims=True) * (1.0 / N)


def _lstm_sage(G, fd, wg, bg, wselfT, wneighT, sb, n1g, n1b, n3g, n3b):
    return _pallas_call(
        _lstm_sage_body,
        grid=(GRID,),
        in_specs=[
            pl.BlockSpec((DEG, TILE, H), lambda i: (0, i, 0)),
            pl.BlockSpec((TILE, H), lambda i: (i, 0)),
            pl.BlockSpec((2 * H, 4 * H), lambda i: (0, 0)),
            pl.BlockSpec((1, 4 * H), lambda i: (0, 0)),
            pl.BlockSpec((H, H), lambda i: (0, 0)),
            pl.BlockSpec((H, H), lambda i: (0, 0)),
            pl.BlockSpec((1, H), lambda i: (0, 0)),
            pl.BlockSpec((1, H), lambda i: (0, 0)),
            pl.BlockSpec((1, H), lambda i: (0, 0)),
            pl.BlockSpec((1, H), lambda i: (0, 0)),
            pl.BlockSpec((1, H), lambda i: (0, 0)),
        ],
        out_specs=[
            pl.BlockSpec((TILE, H), lambda i: (i, 0)),
            pl.BlockSpec((1, H), lambda i: (0, 0)),
        ],
        out_shape=[
            jax.ShapeDtypeStruct((N, H), jnp.float32),
            jax.ShapeDtypeStruct((1, H), jnp.float32),
        ],
    )(G, fd, wg, bg, wselfT, wneighT, sb, n1g, n1b, n3g, n3b)


def _head_body(hg2m_ref, actm_ref, durm_ref, mhW1T_ref, mhb1_ref, mhW2T_ref,
               mhb2_ref, mtW1T_ref, mtb1_ref, mtW2T_ref, mtb2_ref, mlpW1T_ref,
               mlpb1_ref, mlpW2T_ref, mlpb2_ref, clsWT_ref, clsb_ref, out_ref):
    hg2m = hg2m_ref[...]
    hgm = actm_ref[...] + durm_ref[...]
    out1 = jax.nn.relu(_dot(jax.nn.relu(_dot(hg2m, mhW1T_ref[...]) + mhb1_ref[...]),
                            mhW2T_ref[...]) + mhb2_ref[...])
    out2 = jax.nn.relu(_dot(jax.nn.relu(_dot(hgm, mtW1T_ref[...]) + mtb1_ref[...]),
                            mtW2T_ref[...]) + mtb2_ref[...])
    w = mlpW1T_ref[...]
    pre = (_dot(out1, w[:H]) + _dot(hg2m, w[H:2 * H]) +
           _dot(out2, w[2 * H:3 * H]) + _dot(hgm, w[3 * H:]) + mlpb1_ref[...])
    preds = jax.nn.relu(_dot(jax.nn.relu(pre), mlpW2T_ref[...]) + mlpb2_ref[...])
    out_ref[...] = _dot(preds, clsWT_ref[...]) + clsb_ref[...]


def _head(hg2m, actm, durm, mhW1T, mhb1, mhW2T, mhb2, mtW1T, mtb1, mtW2T,
          mtb2, mlpW1T, mlpb1, mlpW2T, mlpb2, clsWT, clsb):
    return _pallas_call(
        _head_body,
        out_shape=jax.ShapeDtypeStruct((1, H), jnp.float32),
    )(hg2m, actm, durm, mhW1T, mhb1, mhW2T, mhb2, mtW1T, mtb1, mtW2T, mtb2,
      mlpW1T, mlpb1, mlpW2T, mlpb2, clsWT, clsb)


C_EDGE = 200
C_EMB = 120
N_PAD = 10240
EP = DEG * N_PAD


def kernel(activity_ids, duration_ids, g2_activity_ids, src_a2d, src_d2a,
           g2_src, emb_activity, emb_duration, emb_activity2, transform_W,
           transform_b, lstm_Wih, lstm_Whh, lstm_bih, lstm_bhh, fc_self_W,
           fc_neigh_W, sage_b, norm1_g, norm1_b, norm3_g, norm3_b, ggc_We,
           gru_Wi, gru_Wh, gru_bi, gru_bh, homo_ln_g, homo_ln_b, mh_W1, mh_b1,
           mh_W2, mh_b2, mt_W1, mt_b1, mt_W2, mt_b2, mlp_W1, mlp_b1, mlp_W2,
           mlp_b2, cls_W, cls_b):
    f32 = jnp.float32

    def prep_edge_idx(src):
        return src.astype(jnp.int32).reshape(N, DEG).T.reshape(E)

    def pad_ids(ids, off):
        return jnp.pad(ids.astype(jnp.int32), (0, N_PAD - N)) + off

    emb_table = jnp.concatenate([emb_activity, emb_duration, emb_activity2], axis=0)
    emb_idx = jnp.concatenate([
        pad_ids(activity_ids, 0),
        pad_ids(duration_ids, 1000),
        pad_ids(g2_activity_ids, 2000),
    ])
    emb_out = _gather_rows(emb_table, emb_idx, 3 * N_PAD, C_EMB)
    h_act = emb_out[0:N]
    h_dur = emb_out[N_PAD:N_PAD + N]
    h_g2 = emb_out[2 * N_PAD:2 * N_PAD + N]

    bf16 = jnp.bfloat16
    a2d_idx = prep_edge_idx(src_a2d)
    d2a_idx = prep_edge_idx(src_d2a)
    weT = ggc_We.T.astype(f32)
    wiT = gru_Wi.T
    whT = gru_Wh.T
    bi = gru_bi.reshape(1, 3 * H)
    bh = gru_bh.reshape(1, 3 * H)

    def sage(G, fd, l, r, ti):
        return _lstm_sage(
            G, fd,
            jnp.concatenate([lstm_Wih[l, r].T, lstm_Whh[l, r].T],
                            axis=0).astype(bf16),
            (lstm_bih[l, r] + lstm_bhh[l, r]).reshape(1, 4 * H),
            fc_self_W[l, r].T, fc_neigh_W[l, r].T,
            sage_b[l, r].reshape(1, H),
            norm1_g[l, ti].reshape(1, H), norm1_b[l, ti].reshape(1, H),
            norm3_g[l, ti].reshape(1, H), norm3_b[l, ti].reshape(1, H))

    h_new = _transform(h_g2, h_dur, transform_W.T, transform_b.reshape(1, H))
    g2_nm = jnp.pad(g2_src.astype(jnp.int32).reshape(N, DEG),
                    ((0, N_PAD - N), (0, 0))).reshape(EP)
    rpw_s = (N_PAD // NW) * DEG
    gr = jnp.arange(EP, dtype=jnp.int32)
    dst_abs = ((gr // rpw_s) // NC_SC) * (N_PAD // NW) + (gr % rpw_s) // DEG
    zeros16 = jnp.zeros((N_PAD // NW, H), jnp.float32)
    S1 = _gather_sum(h_new, g2_nm, dst_abs, zeros16, N_PAD)
    Ga0 = _gather_rows(h_act, a2d_idx, E, C_EDGE).reshape(DEG, N, H)
    Gd0 = _gather_rows(h_dur, d2a_idx, E, C_EDGE).reshape(DEG, N, H)
    hh1 = _ggc_step(S1, h_new, weT, wiT, whT, bi, bh)
    S2 = _gather_sum(hh1, g2_nm, dst_abs, zeros16, N_PAD)
    new_dur, _ = sage(Ga0, h_dur, 0, 0, 1)
    new_act, _ = sage(Gd0, h_act, 0, 1, 0)
    Gd1 = _gather_rows(new_dur, d2a_idx, E, C_EDGE).reshape(DEG, N, H)
    Ga1 = _gather_rows(new_act, a2d_idx, E, C_EDGE).reshape(DEG, N, H)
    hg2_mean = _ggc_final(S2, hh1, h_new, weT, wiT, whT, bi, bh,
                          homo_ln_g.reshape(1, H), homo_ln_b.reshape(1, H))
    _, dur_mean = sage(Ga1, new_dur, 1, 0, 1)
    _, act_mean = sage(Gd1, new_act, 1, 1, 0)

    clsWT = jnp.zeros((H, H), f32).at[:, :16].set(cls_W.T)
    clsb = jnp.zeros((1, H), f32).at[0, :16].set(cls_b)
    logits = _head(
        hg2_mean, act_mean, dur_mean,
        mh_W1.T, mh_b1.reshape(1, H), mh_W2.T, mh_b2.reshape(1, H),
        mt_W1.T, mt_b1.reshape(1, H), mt_W2.T, mt_b2.reshape(1, H),
        mlp_W1.T, mlp_b1.reshape(1, 2 * H), mlp_W2.T, mlp_b2.reshape(1, H),
        clsWT, clsb)
    return logits[:, :16]


# speedup vs baseline: 1.1344x; 1.1300x over previous
import functools

import jax
import jax.numpy as jnp
from jax import lax
from jax.experimental import pallas as pl
from jax.experimental.pallas import tpu as pltpu
from jax.experimental.pallas import tpu_sc as plsc

N = 10000
DEG = 16
E = N * DEG
H = 128
NC_SC = 2
NS_SC = 16
NW = NC_SC * NS_SC

_pallas_call = pl.pallas_call


@functools.lru_cache(maxsize=None)
def _make_sc_gather(M, C, dtype, W):
    npw = M // (NW * C)
    assert npw * NW * C == M
    mesh = plsc.VectorSubcoreMesh(core_axis_name="c", subcore_axis_name="s")

    rpw = npw * C
    NB = 3

    def body(table_hbm, idx_hbm, out_hbm, idx_v, *bufsem):
        bufs = bufsem[:NB]
        sgs = bufsem[NB:2 * NB]
        sos = bufsem[2 * NB:]
        wid = lax.axis_index("s") * NC_SC + lax.axis_index("c")
        chunk0 = wid * npw
        pltpu.sync_copy(idx_hbm.at[pl.ds(wid * rpw, rpw)], idx_v)
        hg = [None] * NB
        ho = [None] * NB
        for j in range(min(NB - 1, npw)):
            hg[j] = pltpu.async_copy(
                table_hbm.at[idx_v.at[pl.ds(j * C, C)]], bufs[j], sgs[j])
        for i in range(npw):
            b = i % NB
            hg[b].wait()
            ho[b] = pltpu.async_copy(
                bufs[b], out_hbm.at[pl.ds((chunk0 + i) * C, C)], sos[b])
            j = i + NB - 1
            if j < npw:
                bj = j % NB
                if ho[bj] is not None:
                    ho[bj].wait()
                    ho[bj] = None
                hg[bj] = pltpu.async_copy(
                    table_hbm.at[idx_v.at[pl.ds(j * C, C)]], bufs[bj], sgs[bj])
        for b in range(NB):
            if ho[b] is not None:
                ho[b].wait()

    return pl.kernel(
        body,
        out_type=jax.ShapeDtypeStruct((M, W), dtype),
        mesh=mesh,
        scratch_types=(
            [pltpu.VMEM((rpw,), jnp.int32)] +
            [pltpu.VMEM((C, W), dtype) for _ in range(NB)] +
            [pltpu.SemaphoreType.DMA for _ in range(2 * NB)]
        ),
    )


def _gather_rows(table, idx_flat, M, C):
    return _make_sc_gather(M, C, table.dtype, H)(table, idx_flat)


def _dot(a, b):
    return jnp.dot(a, b, preferred_element_type=jnp.float32)


def _ln(x, g, b):
    mu = jnp.mean(x, axis=-1, keepdims=True)
    d = x - mu
    var = jnp.mean(d * d, axis=-1, keepdims=True)
    return d * lax.rsqrt(var + 1e-5) * g + b


def _leaky(x):
    return jnp.where(x >= 0, x, 0.01 * x)


TILE = 400
GRID = N // TILE


def _transform_body(hg2_ref, hdur_ref, wt_ref, b_ref, out_ref):
    out_ref[...] = (_dot(hg2_ref[...], wt_ref[:H]) +
                    _dot(hdur_ref[...], wt_ref[H:]) + b_ref[...])


def _transform(h_g2, h_dur, wt, b):
    return _pallas_call(
        _transform_body,
        grid=(GRID,),
        in_specs=[
            pl.BlockSpec((TILE, H), lambda i: (i, 0)),
            pl.BlockSpec((TILE, H), lambda i: (i, 0)),
            pl.BlockSpec((2 * H, H), lambda i: (0, 0)),
            pl.BlockSpec((1, H), lambda i: (0, 0)),
        ],
        out_specs=pl.BlockSpec((TILE, H), lambda i: (i, 0)),
        out_shape=jax.ShapeDtypeStruct((N, H), jnp.float32),
    )(h_g2, h_dur, wt, b)


def _gru_math(g2_ref, hh, weT_ref, wiT_ref, whT_ref, bi_ref, bh_ref):
    s = g2_ref[0].astype(jnp.float32)
    for t in range(1, DEG):
        s = s + g2_ref[t].astype(jnp.float32)
    a = _dot(s, weT_ref[...])
    gi = _dot(a, wiT_ref[...]) + bi_ref[...]
    gh = _dot(hh, whT_ref[...]) + bh_ref[...]
    ir, iz, inn = gi[:, :H], gi[:, H:2 * H], gi[:, 2 * H:]
    hr, hz, hn2 = gh[:, :H], gh[:, H:2 * H], gh[:, 2 * H:]
    rg = jax.nn.sigmoid(ir + hr)
    zg = jax.nn.sigmoid(iz + hz)
    ng = jnp.tanh(inn + rg * hn2)
    return (1.0 - zg) * ng + zg * hh


def _ggc_step_body(g2_ref, hh_ref, weT_ref, wiT_ref, whT_ref, bi_ref, bh_ref,
                   out_ref):
    out_ref[...] = _gru_math(g2_ref, hh_ref[...], weT_ref, wiT_ref, whT_ref,
                             bi_ref, bh_ref)


def _ggc_step(G2, hh, weT, wiT, whT, bi, bh):
    return _pallas_call(
        _ggc_step_body,
        grid=(GRID,),
        in_specs=[
            pl.BlockSpec((DEG, TILE, H), lambda i: (0, i, 0)),
            pl.BlockSpec((TILE, H), lambda i: (i, 0)),
            pl.BlockSpec((H, H), lambda i: (0, 0)),
            pl.BlockSpec((H, 3 * H), lambda i: (0, 0)),
            pl.BlockSpec((H, 3 * H), lambda i: (0, 0)),
            pl.BlockSpec((1, 3 * H), lambda i: (0, 0)),
            pl.BlockSpec((1, 3 * H), lambda i: (0, 0)),
        ],
        out_specs=pl.BlockSpec((TILE, H), lambda i: (i, 0)),
        out_shape=jax.ShapeDtypeStruct((N, H), jnp.float32),
    )(G2, hh, weT, wiT, whT, bi, bh)


def _ggc_final_body(g2_ref, hh_ref, res_ref, weT_ref, wiT_ref, whT_ref,
                    bi_ref, bh_ref, lng_ref, lnb_ref, mean_ref):
    hh2 = _gru_math(g2_ref, hh_ref[...], weT_ref, wiT_ref, whT_ref, bi_ref,
                    bh_ref)
    v = _leaky(_ln(hh2 + res_ref[...], lng_ref[...], lnb_ref[...]))

    @pl.when(pl.program_id(0) == 0)
    def _():
        mean_ref[...] = jnp.zeros_like(mean_ref)

    mean_ref[...] += jnp.sum(v, axis=0, keepdims=True) * (1.0 / N)


def _ggc_final(G2, hh, res, weT, wiT, whT, bi, bh, lng, lnb):
    return _pallas_call(
        _ggc_final_body,
        grid=(GRID,),
        in_specs=[
            pl.BlockSpec((DEG, TILE, H), lambda i: (0, i, 0)),
            pl.BlockSpec((TILE, H), lambda i: (i, 0)),
            pl.BlockSpec((TILE, H), lambda i: (i, 0)),
            pl.BlockSpec((H, H), lambda i: (0, 0)),
            pl.BlockSpec((H, 3 * H), lambda i: (0, 0)),
            pl.BlockSpec((H, 3 * H), lambda i: (0, 0)),
            pl.BlockSpec((1, 3 * H), lambda i: (0, 0)),
            pl.BlockSpec((1, 3 * H), lambda i: (0, 0)),
            pl.BlockSpec((1, H), lambda i: (0, 0)),
            pl.BlockSpec((1, H), lambda i: (0, 0)),
        ],
        out_specs=pl.BlockSpec((1, H), lambda i: (0, 0)),
        out_shape=jax.ShapeDtypeStruct((1, H), jnp.float32),
    )(G2, hh, res, weT, wiT, whT, bi, bh, lng, lnb)


def _lstm_sage_body(g_ref, fd_ref, wg_ref, bg_ref, wselfT_ref,
                    wneighT_ref, sb_ref, n1g_ref, n1b_ref, n3g_ref, n3b_ref,
                    out_ref, mean_ref):
    fd = fd_ref[...]
    w = wg_ref[...]
    bg = bg_ref[...]
    h = jnp.zeros((TILE, H), jnp.float32)
    c = jnp.zeros((TILE, H), jnp.float32)
    for t in range(DEG):
        xh = jnp.concatenate(
            [g_ref[t].astype(jnp.bfloat16), h.astype(jnp.bfloat16)], axis=1)
        gates = _dot(xh, w) + bg
        i_ = gates[:, :H]
        f_ = gates[:, H:2 * H]
        g_ = gates[:, 2 * H:3 * H]
        o_ = gates[:, 3 * H:]
        c = jax.nn.sigmoid(f_) * c + jax.nn.sigmoid(i_) * jnp.tanh(g_)
        h = jax.nn.sigmoid(o_) * jnp.tanh(c)
    conv = _dot(fd, wselfT_ref[...]) + _dot(h, wneighT_ref[...]) + sb_ref[...]
    v = _leaky(_ln(conv, n1g_ref[...], n1b_ref[...]))
    v = fd + v
    v = _leaky(_ln(v, n3g_ref[...], n3b_ref[...]))
    out_ref[...] = v

    @pl.when(pl.program_id(0) == 0)
    def _():
        mean_ref[...] = jnp.zeros_like(mean_ref)

    mean_ref[...] += jnp.sum(v, axis=0, keepdims=True) * (1.0 / N)


def _lstm_sage(G, fd, wg, bg, wselfT, wneighT, sb, n1g, n1b, n3g, n3b):
    return _pallas_call(
        _lstm_sage_body,
        grid=(GRID,),
        in_specs=[
            pl.BlockSpec((DEG, TILE, H), lambda i: (0, i, 0)),
            pl.BlockSpec((TILE, H), lambda i: (i, 0)),
            pl.BlockSpec((2 * H, 4 * H), lambda i: (0, 0)),
            pl.BlockSpec((1, 4 * H), lambda i: (0, 0)),
            pl.BlockSpec((H, H), lambda i: (0, 0)),
            pl.BlockSpec((H, H), lambda i: (0, 0)),
            pl.BlockSpec((1, H), lambda i: (0, 0)),
            pl.BlockSpec((1, H), lambda i: (0, 0)),
            pl.BlockSpec((1, H), lambda i: (0, 0)),
            pl.BlockSpec((1, H), lambda i: (0, 0)),
            pl.BlockSpec((1, H), lambda i: (0, 0)),
        ],
        out_specs=[
            pl.BlockSpec((TILE, H), lambda i: (i, 0)),
            pl.BlockSpec((1, H), lambda i: (0, 0)),
        ],
        out_shape=[
            jax.ShapeDtypeStruct((N, H), jnp.float32),
            jax.ShapeDtypeStruct((1, H), jnp.float32),
        ],
    )(G, fd, wg, bg, wselfT, wneighT, sb, n1g, n1b, n3g, n3b)


def _head_body(hg2m_ref, actm_ref, durm_ref, mhW1T_ref, mhb1_ref, mhW2T_ref,
               mhb2_ref, mtW1T_ref, mtb1_ref, mtW2T_ref, mtb2_ref, mlpW1T_ref,
               mlpb1_ref, mlpW2T_ref, mlpb2_ref, clsWT_ref, clsb_ref, out_ref):
    hg2m = hg2m_ref[...]
    hgm = actm_ref[...] + durm_ref[...]
    out1 = jax.nn.relu(_dot(jax.nn.relu(_dot(hg2m, mhW1T_ref[...]) + mhb1_ref[...]),
                            mhW2T_ref[...]) + mhb2_ref[...])
    out2 = jax.nn.relu(_dot(jax.nn.relu(_dot(hgm, mtW1T_ref[...]) + mtb1_ref[...]),
                            mtW2T_ref[...]) + mtb2_ref[...])
    w = mlpW1T_ref[...]
    pre = (_dot(out1, w[:H]) + _dot(hg2m, w[H:2 * H]) +
           _dot(out2, w[2 * H:3 * H]) + _dot(hgm, w[3 * H:]) + mlpb1_ref[...])
    preds = jax.nn.relu(_dot(jax.nn.relu(pre), mlpW2T_ref[...]) + mlpb2_ref[...])
    out_ref[...] = _dot(preds, clsWT_ref[...]) + clsb_ref[...]


def _head(hg2m, actm, durm, mhW1T, mhb1, mhW2T, mhb2, mtW1T, mtb1, mtW2T,
          mtb2, mlpW1T, mlpb1, mlpW2T, mlpb2, clsWT, clsb):
    return _pallas_call(
        _head_body,
        out_shape=jax.ShapeDtypeStruct((1, H), jnp.float32),
    )(hg2m, actm, durm, mhW1T, mhb1, mhW2T, mhb2, mtW1T, mtb1, mtW2T, mtb2,
      mlpW1T, mlpb1, mlpW2T, mlpb2, clsWT, clsb)


C_EDGE = 200
C_EMB = 120
N_PAD = 10240


def kernel(activity_ids, duration_ids, g2_activity_ids, src_a2d, src_d2a,
           g2_src, emb_activity, emb_duration, emb_activity2, transform_W,
           transform_b, lstm_Wih, lstm_Whh, lstm_bih, lstm_bhh, fc_self_W,
           fc_neigh_W, sage_b, norm1_g, norm1_b, norm3_g, norm3_b, ggc_We,
           gru_Wi, gru_Wh, gru_bi, gru_bh, homo_ln_g, homo_ln_b, mh_W1, mh_b1,
           mh_W2, mh_b2, mt_W1, mt_b1, mt_W2, mt_b2, mlp_W1, mlp_b1, mlp_W2,
           mlp_b2, cls_W, cls_b):
    f32 = jnp.float32

    def prep_edge_idx(src):
        return src.astype(jnp.int32).reshape(N, DEG).T.reshape(E)

    def pad_ids(ids, off):
        return jnp.pad(ids.astype(jnp.int32), (0, N_PAD - N)) + off

    emb_table = jnp.concatenate([emb_activity, emb_duration, emb_activity2], axis=0)
    emb_idx = jnp.concatenate([
        pad_ids(activity_ids, 0),
        pad_ids(duration_ids, 1000),
        pad_ids(g2_activity_ids, 2000),
    ])
    emb_out = _gather_rows(emb_table, emb_idx, 3 * N_PAD, C_EMB)
    h_act = emb_out[0:N]
    h_dur = emb_out[N_PAD:N_PAD + N]
    h_g2 = emb_out[2 * N_PAD:2 * N_PAD + N]

    bf16 = jnp.bfloat16
    g2_idx = prep_edge_idx(g2_src)
    a2d_idx = prep_edge_idx(src_a2d)
    d2a_idx = prep_edge_idx(src_d2a)
    weT = ggc_We.T.astype(f32)
    wiT = gru_Wi.T
    whT = gru_Wh.T
    bi = gru_bi.reshape(1, 3 * H)
    bh = gru_bh.reshape(1, 3 * H)

    def sage(G, fd, l, r, ti):
        return _lstm_sage(
            G, fd,
            jnp.concatenate([lstm_Wih[l, r].T, lstm_Whh[l, r].T],
                            axis=0).astype(bf16),
            (lstm_bih[l, r] + lstm_bhh[l, r]).reshape(1, 4 * H),
            fc_self_W[l, r].T, fc_neigh_W[l, r].T,
            sage_b[l, r].reshape(1, H),
            norm1_g[l, ti].reshape(1, H), norm1_b[l, ti].reshape(1, H),
            norm3_g[l, ti].reshape(1, H), norm3_b[l, ti].reshape(1, H))

    h_new = _transform(h_g2, h_dur, transform_W.T, transform_b.reshape(1, H))
    G2 = _gather_rows(h_new, g2_idx, E, C_EDGE).reshape(DEG, N, H)
    Ga0 = _gather_rows(h_act, a2d_idx, E, C_EDGE).reshape(DEG, N, H)
    Gd0 = _gather_rows(h_dur, d2a_idx, E, C_EDGE).reshape(DEG, N, H)
    hh1 = _ggc_step(G2, h_new, weT, wiT, whT, bi, bh)
    G2b = _gather_rows(hh1, g2_idx, E, C_EDGE).reshape(DEG, N, H)
    new_dur, _ = sage(Ga0, h_dur, 0, 0, 1)
    new_act, _ = sage(Gd0, h_act, 0, 1, 0)
    Gd1 = _gather_rows(new_dur, d2a_idx, E, C_EDGE).reshape(DEG, N, H)
    Ga1 = _gather_rows(new_act, a2d_idx, E, C_EDGE).reshape(DEG, N, H)
    hg2_mean = _ggc_final(G2b, hh1, h_new, weT, wiT, whT, bi, bh,
                          homo_ln_g.reshape(1, H), homo_ln_b.reshape(1, H))
    _, dur_mean = sage(Ga1, new_dur, 1, 0, 1)
    _, act_mean = sage(Gd1, new_act, 1, 1, 0)

    clsWT = jnp.zeros((H, H), f32).at[:, :16].set(cls_W.T)
    clsb = jnp.zeros((1, H), f32).at[0, :16].set(cls_b)
    logits = _head(
        hg2_mean, act_mean, dur_mean,
        mh_W1.T, mh_b1.reshape(1, H), mh_W2.T, mh_b2.reshape(1, H),
        mt_W1.T, mt_b1.reshape(1, H), mt_W2.T, mt_b2.reshape(1, H),
        mlp_W1.T, mlp_b1.reshape(1, 2 * H), mlp_W2.T, mlp_b2.reshape(1, H),
        clsWT, clsb)
    return logits[:, :16]
